# baseline (speedup 1.0000x reference)
import functools

import jax
import jax.numpy as jnp
from jax import lax
from jax.experimental import pallas as pl
from jax.experimental.pallas import tpu as pltpu
from jax.experimental.pallas import tpu_sc as plsc

_D = 128
_H = 8
_HD = 16
_ROW = 144
_NTILES = 32


def _proj_body(x_ref, w_ref, b_ref, o_ref):
    o_ref[...] = (
        jnp.dot(x_ref[...], w_ref[...], preferred_element_type=jnp.float32)
        + b_ref[...]
    )


def _project(x, w, b, blk):
    n, d_in = x.shape
    d_out = w.shape[1]
    return pl.pallas_call(
        _proj_body,
        grid=(n // blk,),
        in_specs=[
            pl.BlockSpec((blk, d_in), lambda i: (i, 0)),
            pl.BlockSpec((d_in, d_out), lambda i: (0, 0)),
            pl.BlockSpec((1, d_out), lambda i: (0, 0)),
        ],
        out_specs=pl.BlockSpec((blk, d_out), lambda i: (i, 0)),
        out_shape=jax.ShapeDtypeStruct((n, d_out), jnp.float32),
    )(x, w, b.reshape(1, d_out))


def _make_sc_gather_kernel(E, C):
    ept = E // _NTILES
    n_chunks = ept // C
    mesh = plsc.VectorSubcoreMesh(core_axis_name="c", subcore_axis_name="s")

    @functools.partial(
        pl.kernel,
        out_type=(
            jax.ShapeDtypeStruct((E, _D), jnp.float32),
            jax.ShapeDtypeStruct((E, _D), jnp.float32),
        ),
        mesh=mesh,
        compiler_params=pltpu.CompilerParams(
            needs_layout_passes=False, use_tc_tiling_on_sc=False),
        scratch_types=[
            pltpu.VMEM((C,), jnp.int32),
            pltpu.VMEM((C,), jnp.int32),
            pltpu.VMEM((C, _D), jnp.float32),
            pltpu.VMEM((C, _D), jnp.float32),
            pltpu.SemaphoreType.DMA,
            pltpu.SemaphoreType.DMA,
        ],
    )
    def k(h_hbm, snd_hbm, rcv_hbm, hs_out, hr_out,
          snd_v, rcv_v, hs_v, hr_v, sem1, sem2):
        cid = lax.axis_index("c")
        sid = lax.axis_index("s")
        base = (cid * 16 + sid) * ept

        def chunk_body(i, carry):
            off = base + i * C
            pltpu.sync_copy(snd_hbm.at[pl.ds(off, C)], snd_v)
            pltpu.sync_copy(rcv_hbm.at[pl.ds(off, C)], rcv_v)
            cp1 = pltpu.async_copy(h_hbm.at[snd_v], hs_v, sem1)
            cp2 = pltpu.async_copy(h_hbm.at[rcv_v], hr_v, sem2)
            cp1.wait()
            cp2.wait()
            pltpu.sync_copy(hs_v, hs_out.at[pl.ds(off, C)])
            pltpu.sync_copy(hr_v, hr_out.at[pl.ds(off, C)])
            return carry

        lax.fori_loop(0, n_chunks, chunk_body, 0)

    return k


def _edge_body(hs_ref, hr_ref, ef_ref, we_ref, be_ref, ad_ref, o_ref, ow_ref):
    hs = hs_ref[...]
    x = (hs + hr_ref[...] + be_ref[...]
         + jnp.dot(ef_ref[...], we_ref[...],
                   preferred_element_type=jnp.float32))
    t = jnp.exp(jnp.minimum(x, 20.0))
    u = t + 1.0
    sq = u * u
    m = x * ((sq - 1.0) / (sq + 1.0))
    logits = jnp.dot(m, ad_ref[...], preferred_element_type=jnp.float32)
    w = jnp.exp(jnp.minimum(logits, 60.0))
    col = lax.broadcasted_iota(jnp.int32, (_H, _D), 1) // _HD
    row = lax.broadcasted_iota(jnp.int32, (_H, _D), 0)
    expand = (col == row).astype(jnp.float32)
    w128 = jnp.dot(w, expand, preferred_element_type=jnp.float32)
    blk = hs.shape[0]
    o_ref[...] = w128 * hs
    ow_ref[...] = jnp.concatenate(
        [w, jnp.zeros((blk, _HD - _H), jnp.float32)], axis=1)


def _edge_compute(hs, hr, ef, we, be, ad, blk, ef_blk_off):
    E = hs.shape[0]
    de = ef.shape[1]
    return pl.pallas_call(
        _edge_body,
        grid=(E // blk,),
        in_specs=[
            pl.BlockSpec((blk, _D), lambda i: (i, 0)),
            pl.BlockSpec((blk, _D), lambda i: (i, 0)),
            pl.BlockSpec((blk, de), lambda i: (i + ef_blk_off, 0)),
            pl.BlockSpec((de, _D), lambda i: (0, 0)),
            pl.BlockSpec((1, _D), lambda i: (0, 0)),
            pl.BlockSpec((_D, _H), lambda i: (0, 0)),
        ],
        out_specs=[
            pl.BlockSpec((blk, _D), lambda i: (i, 0)),
            pl.BlockSpec((blk, _HD), lambda i: (i, 0)),
        ],
        out_shape=[
            jax.ShapeDtypeStruct((E, _D), jnp.float32),
            jax.ShapeDtypeStruct((E, _HD), jnp.float32),
        ],
    )(hs, hr, ef, we, be.reshape(1, _D), ad)


def _make_sc_scatter_kernel(E, N, C):
    ept = E // _NTILES
    n_chunks = ept // C
    npad = -(-N // 16) * 16
    rpt = npad // 16
    mesh = plsc.VectorSubcoreMesh(core_axis_name="c", subcore_axis_name="s")

    @functools.partial(
        pl.kernel,
        out_type=(
            jax.ShapeDtypeStruct((2, npad, _D), jnp.float32),
            jax.ShapeDtypeStruct((2, npad, _HD), jnp.float32),
        ),
        mesh=mesh,
        compiler_params=pltpu.CompilerParams(
            needs_layout_passes=False, use_tc_tiling_on_sc=False),
        scratch_types=[
            pltpu.VMEM((C,), jnp.int32),
            pltpu.VMEM((C, _D), jnp.float32),
            pltpu.VMEM((C, _HD), jnp.float32),
            pltpu.VMEM_SHARED((npad, _D), jnp.float32),
            pltpu.VMEM_SHARED((npad, _HD), jnp.float32),
        ],
    )
    def k(msg_hbm, w_hbm, rcv_hbm, zm_hbm, zw_hbm, outm_hbm, outw_hbm,
          rcv_v, msg_v, w_v, accm, accw):
        cid = lax.axis_index("c")
        sid = lax.axis_index("s")
        r0 = sid * rpt
        pltpu.sync_copy(zm_hbm.at[pl.ds(r0, rpt)], accm.at[pl.ds(r0, rpt)])
        pltpu.sync_copy(zw_hbm.at[pl.ds(r0, rpt)], accw.at[pl.ds(r0, rpt)])
        plsc.subcore_barrier()

        base = (cid * 16 + sid) * ept

        def chunk_body(i, carry):
            off = base + i * C
            pltpu.sync_copy(rcv_hbm.at[pl.ds(off, C)], rcv_v)
            pltpu.sync_copy(msg_hbm.at[pl.ds(off, C)], msg_v)
            pltpu.sync_copy(w_hbm.at[pl.ds(off, C)], w_v)
            pltpu.sync_copy(msg_v, accm.at[rcv_v], add=True)
            pltpu.sync_copy(w_v, accw.at[rcv_v], add=True)
            return carry

        lax.fori_loop(0, n_chunks, chunk_body, 0)
        plsc.subcore_barrier()
        pltpu.sync_copy(accm.at[pl.ds(r0, rpt)],
                        outm_hbm.at[cid, pl.ds(r0, rpt)])
        pltpu.sync_copy(accw.at[pl.ds(r0, rpt)],
                        outw_hbm.at[cid, pl.ds(r0, rpt)])

    return k


def _combine_body(*refs):
    o_ref = refs[-1]
    k = (len(refs) - 1) // 2
    msg = sum(r[0] + r[1] for r in refs[:k])
    sw = sum(r[0] + r[1] for r in refs[k:2 * k])
    den = sw[:, :_H]
    col = lax.broadcasted_iota(jnp.int32, (_H, _D), 1) // _HD
    row = lax.broadcasted_iota(jnp.int32, (_H, _D), 0)
    expand = (col == row).astype(jnp.float32)
    denb = jnp.dot(den, expand, preferred_element_type=jnp.float32)
    o_ref[...] = jnp.where(denb > 0.0, msg / denb, 0.0)


def _combine(pms, pws, N, blk):
    k = len(pms)
    return pl.pallas_call(
        _combine_body,
        grid=(N // blk,),
        in_specs=(
            [pl.BlockSpec((2, blk, _D), lambda i: (0, i, 0))] * k
            + [pl.BlockSpec((2, blk, _HD), lambda i: (0, i, 0))] * k
        ),
        out_specs=pl.BlockSpec((blk, _D), lambda i: (i, 0)),
        out_shape=jax.ShapeDtypeStruct((N, _D), jnp.float32),
    )(*pms, *pws)


def kernel(node_features, senders, receivers, edge_features, W_kernel,
           W_bias, We_kernel, We_bias, a_kernel):
    N = node_features.shape[0]
    E = senders.shape[0]
    snd = senders.astype(jnp.int32)
    rcv = receivers.astype(jnp.int32)
    h = _project(node_features, W_kernel, W_bias, 1000)
    ad = (jnp.repeat(jnp.eye(_H, dtype=jnp.float32), _HD, axis=0)
          * a_kernel.reshape(_D, 1))
    npad = -(-N // 16) * 16
    zm = jnp.zeros((npad, _D), jnp.float32)
    zw = jnp.zeros((npad, _HD), jnp.float32)

    blk = 2000
    sizes = (64000, 64000, 64000, 64000, 64000)
    gathered = []
    off = 0
    for sz in sizes:
        g = _make_sc_gather_kernel(sz, 200)
        hs_c, hr_c = g(h, snd[off:off + sz], rcv[off:off + sz])
        gathered.append((off, sz, hs_c, hr_c))
        off += sz
    pms, pws = [], []
    for off, sz, hs_c, hr_c in gathered:
        msg_c, w_c = _edge_compute(hs_c, hr_c, edge_features, We_kernel,
                                   We_bias, ad, blk, off // blk)
        pm, pw = _make_sc_scatter_kernel(sz, N, 200)(
            msg_c, w_c, rcv[off:off + sz], zm, zw)
        pms.append(pm)
        pws.append(pw)
    return _combine(pms, pws, N, 1000)


# speedup vs baseline: 1.0498x; 1.0498x over previous
import functools

import jax
import jax.numpy as jnp
from jax import lax
from jax.experimental import pallas as pl
from jax.experimental.pallas import tpu as pltpu
from jax.experimental.pallas import tpu_sc as plsc

_D = 128
_H = 8
_HD = 16
_ROW = 144
_NTILES = 32


def _proj_body(x_ref, w_ref, b_ref, o_ref):
    o_ref[...] = (
        jnp.dot(x_ref[...], w_ref[...], preferred_element_type=jnp.float32)
        + b_ref[...]
    )


def _project(x, w, b, blk):
    n, d_in = x.shape
    d_out = w.shape[1]
    return pl.pallas_call(
        _proj_body,
        grid=(n // blk,),
        in_specs=[
            pl.BlockSpec((blk, d_in), lambda i: (i, 0)),
            pl.BlockSpec((d_in, d_out), lambda i: (0, 0)),
            pl.BlockSpec((1, d_out), lambda i: (0, 0)),
        ],
        out_specs=pl.BlockSpec((blk, d_out), lambda i: (i, 0)),
        out_shape=jax.ShapeDtypeStruct((n, d_out), jnp.float32),
    )(x, w, b.reshape(1, d_out))


def _make_sc_gather_kernel(E, C):
    ept = E // _NTILES
    n_chunks = ept // C
    mesh = plsc.VectorSubcoreMesh(core_axis_name="c", subcore_axis_name="s")

    @functools.partial(
        pl.kernel,
        out_type=(
            jax.ShapeDtypeStruct((E, _D), jnp.float32),
            jax.ShapeDtypeStruct((E, _D), jnp.float32),
        ),
        mesh=mesh,
        compiler_params=pltpu.CompilerParams(
            needs_layout_passes=False, use_tc_tiling_on_sc=False),
        scratch_types=[
            pltpu.VMEM((C,), jnp.int32),
            pltpu.VMEM((C,), jnp.int32),
            pltpu.VMEM((C, _D), jnp.float32),
            pltpu.VMEM((C, _D), jnp.float32),
            pltpu.SemaphoreType.DMA,
            pltpu.SemaphoreType.DMA,
        ],
    )
    def k(h_hbm, snd_hbm, rcv_hbm, hs_out, hr_out,
          snd_v, rcv_v, hs_v, hr_v, sem1, sem2):
        cid = lax.axis_index("c")
        sid = lax.axis_index("s")
        base = (cid * 16 + sid) * ept

        def chunk_body(i, carry):
            off = base + i * C
            pltpu.sync_copy(snd_hbm.at[pl.ds(off, C)], snd_v)
            pltpu.sync_copy(rcv_hbm.at[pl.ds(off, C)], rcv_v)
            cp1 = pltpu.async_copy(h_hbm.at[snd_v], hs_v, sem1)
            cp2 = pltpu.async_copy(h_hbm.at[rcv_v], hr_v, sem2)
            cp1.wait()
            cp2.wait()
            pltpu.sync_copy(hs_v, hs_out.at[pl.ds(off, C)])
            pltpu.sync_copy(hr_v, hr_out.at[pl.ds(off, C)])
            return carry

        lax.fori_loop(0, n_chunks, chunk_body, 0)

    return k


def _edge_body(hs_ref, hr_ref, ef_ref, we_ref, be_ref, ad_ref, o_ref, ow_ref):
    hs = hs_ref[...]
    x = (hs + hr_ref[...] + be_ref[...]
         + jnp.dot(ef_ref[...], we_ref[...],
                   preferred_element_type=jnp.float32))
    t = jnp.exp(jnp.minimum(x, 20.0))
    u = t + 1.0
    sq = u * u
    m = x * ((sq - 1.0) / (sq + 1.0))
    logits = jnp.dot(m, ad_ref[...], preferred_element_type=jnp.float32)
    w = jnp.exp(jnp.minimum(logits, 60.0))
    col = lax.broadcasted_iota(jnp.int32, (_H, _D), 1) // _HD
    row = lax.broadcasted_iota(jnp.int32, (_H, _D), 0)
    expand = (col == row).astype(jnp.float32)
    w128 = jnp.dot(w, expand, preferred_element_type=jnp.float32)
    blk = hs.shape[0]
    o_ref[...] = w128 * hs
    ow_ref[...] = jnp.concatenate(
        [w, jnp.zeros((blk, _HD - _H), jnp.float32)], axis=1)


def _edge_compute(hs, hr, ef, we, be, ad, blk, ef_blk_off):
    E = hs.shape[0]
    de = ef.shape[1]
    return pl.pallas_call(
        _edge_body,
        grid=(E // blk,),
        in_specs=[
            pl.BlockSpec((blk, _D), lambda i: (i, 0)),
            pl.BlockSpec((blk, _D), lambda i: (i, 0)),
            pl.BlockSpec((blk, de), lambda i: (i + ef_blk_off, 0)),
            pl.BlockSpec((de, _D), lambda i: (0, 0)),
            pl.BlockSpec((1, _D), lambda i: (0, 0)),
            pl.BlockSpec((_D, _H), lambda i: (0, 0)),
        ],
        out_specs=[
            pl.BlockSpec((blk, _D), lambda i: (i, 0)),
            pl.BlockSpec((blk, _HD), lambda i: (i, 0)),
        ],
        out_shape=[
            jax.ShapeDtypeStruct((E, _D), jnp.float32),
            jax.ShapeDtypeStruct((E, _HD), jnp.float32),
        ],
    )(hs, hr, ef, we, be.reshape(1, _D), ad)


def _make_sc_scatter_kernel(E, N, C):
    ept = E // _NTILES
    n_chunks = ept // C
    npad = -(-N // 16) * 16
    rpt = npad // 16
    mesh = plsc.VectorSubcoreMesh(core_axis_name="c", subcore_axis_name="s")

    @functools.partial(
        pl.kernel,
        out_type=(
            jax.ShapeDtypeStruct((2, npad, _D), jnp.float32),
            jax.ShapeDtypeStruct((2, npad, _HD), jnp.float32),
        ),
        mesh=mesh,
        compiler_params=pltpu.CompilerParams(
            needs_layout_passes=False, use_tc_tiling_on_sc=False),
        scratch_types=[
            pltpu.VMEM((C,), jnp.int32),
            pltpu.VMEM((C, _D), jnp.float32),
            pltpu.VMEM((C, _HD), jnp.float32),
            pltpu.VMEM_SHARED((npad, _D), jnp.float32),
            pltpu.VMEM_SHARED((npad, _HD), jnp.float32),
        ],
    )
    def k(msg_hbm, w_hbm, rcv_hbm, zm_hbm, zw_hbm, outm_hbm, outw_hbm,
          rcv_v, msg_v, w_v, accm, accw):
        cid = lax.axis_index("c")
        sid = lax.axis_index("s")
        r0 = sid * rpt
        pltpu.sync_copy(zm_hbm.at[pl.ds(r0, rpt)], accm.at[pl.ds(r0, rpt)])
        pltpu.sync_copy(zw_hbm.at[pl.ds(r0, rpt)], accw.at[pl.ds(r0, rpt)])
        plsc.subcore_barrier()

        base = (cid * 16 + sid) * ept

        def chunk_body(i, carry):
            off = base + i * C
            pltpu.sync_copy(rcv_hbm.at[pl.ds(off, C)], rcv_v)
            pltpu.sync_copy(msg_hbm.at[pl.ds(off, C)], msg_v)
            pltpu.sync_copy(w_hbm.at[pl.ds(off, C)], w_v)
            pltpu.sync_copy(msg_v, accm.at[rcv_v], add=True)
            pltpu.sync_copy(w_v, accw.at[rcv_v], add=True)
            return carry

        lax.fori_loop(0, n_chunks, chunk_body, 0)
        plsc.subcore_barrier()
        pltpu.sync_copy(accm.at[pl.ds(r0, rpt)],
                        outm_hbm.at[cid, pl.ds(r0, rpt)])
        pltpu.sync_copy(accw.at[pl.ds(r0, rpt)],
                        outw_hbm.at[cid, pl.ds(r0, rpt)])

    return k


def _combine_body(*refs):
    o_ref = refs[-1]
    k = (len(refs) - 1) // 2
    msg = sum(r[0] + r[1] for r in refs[:k])
    sw = sum(r[0] + r[1] for r in refs[k:2 * k])
    den = sw[:, :_H]
    col = lax.broadcasted_iota(jnp.int32, (_H, _D), 1) // _HD
    row = lax.broadcasted_iota(jnp.int32, (_H, _D), 0)
    expand = (col == row).astype(jnp.float32)
    denb = jnp.dot(den, expand, preferred_element_type=jnp.float32)
    o_ref[...] = jnp.where(denb > 0.0, msg / denb, 0.0)


def _combine(pms, pws, N, blk):
    k = len(pms)
    return pl.pallas_call(
        _combine_body,
        grid=(N // blk,),
        in_specs=(
            [pl.BlockSpec((2, blk, _D), lambda i: (0, i, 0))] * k
            + [pl.BlockSpec((2, blk, _HD), lambda i: (0, i, 0))] * k
        ),
        out_specs=pl.BlockSpec((blk, _D), lambda i: (i, 0)),
        out_shape=jax.ShapeDtypeStruct((N, _D), jnp.float32),
    )(*pms, *pws)


def kernel(node_features, senders, receivers, edge_features, W_kernel,
           W_bias, We_kernel, We_bias, a_kernel):
    N = node_features.shape[0]
    E = senders.shape[0]
    snd = senders.astype(jnp.int32)
    rcv = receivers.astype(jnp.int32)
    h = _project(node_features, W_kernel, W_bias, 1000)
    ad = (jnp.repeat(jnp.eye(_H, dtype=jnp.float32), _HD, axis=0)
          * a_kernel.reshape(_D, 1))
    npad = -(-N // 16) * 16
    zm = jnp.zeros((npad, _D), jnp.float32)
    zw = jnp.zeros((npad, _HD), jnp.float32)

    blk = 2000
    sizes = (128000, 96000, 96000)
    gathered = []
    off = 0
    for sz in sizes:
        g = _make_sc_gather_kernel(sz, 200)
        hs_c, hr_c = g(h, snd[off:off + sz], rcv[off:off + sz])
        gathered.append((off, sz, hs_c, hr_c))
        off += sz
    pms, pws = [], []
    for off, sz, hs_c, hr_c in gathered:
        msg_c, w_c = _edge_compute(hs_c, hr_c, edge_features, We_kernel,
                                   We_bias, ad, blk, off // blk)
        pm, pw = _make_sc_scatter_kernel(sz, N, 200)(
            msg_c, w_c, rcv[off:off + sz], zm, zw)
        pms.append(pm)
        pws.append(pw)
    return _combine(pms, pws, N, 1000)
